# baseline (speedup 1.0000x reference)
import functools

import jax
import jax.numpy as jnp
import numpy as np
from jax import lax
from jax.experimental import pallas as pl
from jax.experimental.pallas import tpu as pltpu

_BS = 256


def _threefry_bits_u32(x1):
    ks0 = jnp.uint32(0)
    ks1 = jnp.uint32(42)
    ks2 = jnp.uint32(0x1BD11BDA) ^ ks0 ^ ks1
    x0 = jnp.zeros_like(x1) + ks0
    x1 = x1 + ks1

    def rounds(x0, x1, rots):
        for d in rots:
            x0 = x0 + x1
            x1 = (x1 << d) | (x1 >> (32 - d))
            x1 = x1 ^ x0
        return x0, x1

    r_a = (13, 15, 26, 6)
    r_b = (17, 29, 16, 24)
    for i, (a0, a1, rots) in enumerate(
        [(ks1, ks2, r_a), (ks2, ks0, r_b), (ks0, ks1, r_a),
         (ks1, ks2, r_b), (ks2, ks0, r_a)]
    ):
        x0, x1 = rounds(x0, x1, rots)
        x0 = x0 + a0
        x1 = x1 + a1 + jnp.uint32(i + 1)
    return x0 ^ x1


def _body(n, bs, bi_ref, bj_ref, probs_ref, out_ref, scratch_ref):
    p = pl.program_id(0)
    k = pl.program_id(1)
    bi = bi_ref[p]
    bj = bj_ref[p]

    @pl.when(k == 0)
    def _compute():
        rl = lax.broadcasted_iota(jnp.int32, (bs, bs), 0)
        cl = lax.broadcasted_iota(jnp.int32, (bs, bs), 1)
        r = rl + bi * bs
        c = cl + bj * bs
        flat = (r * n + c).astype(jnp.uint32)
        bits = _threefry_bits_u32(flat)
        fbits = (bits >> 9) | jnp.uint32(0x3F800000)
        noise = lax.bitcast_convert_type(fbits, jnp.float32) - 1.0
        bern = (noise < probs_ref[...]).astype(jnp.float32)
        bern_t = bern.T
        diag = bi == bj
        lower = rl > cl
        out_ref[...] = jnp.where(diag & lower, bern_t, bern)
        scratch_ref[...] = jnp.where(diag & jnp.logical_not(lower), bern, bern_t)

    @pl.when(k == 1)
    def _mirror():
        out_ref[...] = scratch_ref[...]


def kernel(edge_probs):
    n = edge_probs.shape[0]
    bs = _BS
    nb = n // bs
    pairs = [(i, j) for i in range(nb) for j in range(i, nb)]
    bi_arr = jnp.asarray(np.array([ij[0] for ij in pairs], dtype=np.int32))
    bj_arr = jnp.asarray(np.array([ij[1] for ij in pairs], dtype=np.int32))
    num_pairs = len(pairs)

    grid_spec = pltpu.PrefetchScalarGridSpec(
        num_scalar_prefetch=2,
        grid=(num_pairs, 2),
        in_specs=[
            pl.BlockSpec((bs, bs), lambda p, k, bi, bj: (bi[p], bj[p])),
        ],
        out_specs=pl.BlockSpec(
            (bs, bs),
            lambda p, k, bi, bj: (
                jnp.where(k == 0, bi[p], bj[p]),
                jnp.where(k == 0, bj[p], bi[p]),
            ),
        ),
        scratch_shapes=[pltpu.VMEM((bs, bs), jnp.float32)],
    )
    return pl.pallas_call(
        functools.partial(_body, n, bs),
        grid_spec=grid_spec,
        out_shape=jax.ShapeDtypeStruct((n, n), jnp.float32),
        compiler_params=pltpu.CompilerParams(
            dimension_semantics=("parallel", "arbitrary"),
        ),
    )(bi_arr, bj_arr, edge_probs)


# speedup vs baseline: 1.4382x; 1.4382x over previous
import functools

import jax
import jax.numpy as jnp
import numpy as np
from jax import lax
from jax.experimental import pallas as pl
from jax.experimental.pallas import tpu as pltpu

_BS = 512


def _threefry_bits_u32(x1):
    ks0 = jnp.uint32(0)
    ks1 = jnp.uint32(42)
    ks2 = jnp.uint32(0x1BD11BDA) ^ ks0 ^ ks1
    x0 = jnp.zeros_like(x1) + ks0
    x1 = x1 + ks1

    def rounds(x0, x1, rots):
        for d in rots:
            x0 = x0 + x1
            x1 = (x1 << d) | (x1 >> (32 - d))
            x1 = x1 ^ x0
        return x0, x1

    r_a = (13, 15, 26, 6)
    r_b = (17, 29, 16, 24)
    for i, (a0, a1, rots) in enumerate(
        [(ks1, ks2, r_a), (ks2, ks0, r_b), (ks0, ks1, r_a),
         (ks1, ks2, r_b), (ks2, ks0, r_a)]
    ):
        x0, x1 = rounds(x0, x1, rots)
        x0 = x0 + a0
        x1 = x1 + a1 + jnp.uint32(i + 1)
    return x0 ^ x1


def _body(n, bs, bi_ref, bj_ref, probs_ref, out_ref, scratch_ref):
    p = pl.program_id(0)
    k = pl.program_id(1)
    bi = bi_ref[p]
    bj = bj_ref[p]

    @pl.when(k == 0)
    def _compute():
        rl = lax.broadcasted_iota(jnp.int32, (bs, bs), 0)
        cl = lax.broadcasted_iota(jnp.int32, (bs, bs), 1)
        r = rl + bi * bs
        c = cl + bj * bs
        flat = (r * n + c).astype(jnp.uint32)
        bits = _threefry_bits_u32(flat)
        fbits = (bits >> 9) | jnp.uint32(0x3F800000)
        noise = lax.bitcast_convert_type(fbits, jnp.float32) - 1.0
        bern = (noise < probs_ref[...]).astype(jnp.float32)
        bern_t = bern.T
        diag = bi == bj
        lower = rl > cl
        out_ref[...] = jnp.where(diag & lower, bern_t, bern)
        scratch_ref[...] = jnp.where(diag & jnp.logical_not(lower), bern, bern_t)

    @pl.when(k == 1)
    def _mirror():
        out_ref[...] = scratch_ref[...]


def kernel(edge_probs):
    n = edge_probs.shape[0]
    bs = _BS
    nb = n // bs
    pairs = [(i, j) for i in range(nb) for j in range(i, nb)]
    bi_arr = jnp.asarray(np.array([ij[0] for ij in pairs], dtype=np.int32))
    bj_arr = jnp.asarray(np.array([ij[1] for ij in pairs], dtype=np.int32))
    num_pairs = len(pairs)

    grid_spec = pltpu.PrefetchScalarGridSpec(
        num_scalar_prefetch=2,
        grid=(num_pairs, 2),
        in_specs=[
            pl.BlockSpec((bs, bs), lambda p, k, bi, bj: (bi[p], bj[p])),
        ],
        out_specs=pl.BlockSpec(
            (bs, bs),
            lambda p, k, bi, bj: (
                jnp.where(k == 0, bi[p], bj[p]),
                jnp.where(k == 0, bj[p], bi[p]),
            ),
        ),
        scratch_shapes=[pltpu.VMEM((bs, bs), jnp.float32)],
    )
    return pl.pallas_call(
        functools.partial(_body, n, bs),
        grid_spec=grid_spec,
        out_shape=jax.ShapeDtypeStruct((n, n), jnp.float32),
        compiler_params=pltpu.CompilerParams(
            dimension_semantics=("parallel", "arbitrary"),
        ),
    )(bi_arr, bj_arr, edge_probs)


# speedup vs baseline: 1.5645x; 1.0878x over previous
import functools

import jax
import jax.numpy as jnp
import numpy as np
from jax import lax
from jax.experimental import pallas as pl
from jax.experimental.pallas import tpu as pltpu

_BS = 1024


def _threefry_bits_u32(x1):
    ks0 = jnp.uint32(0)
    ks1 = jnp.uint32(42)
    ks2 = jnp.uint32(0x1BD11BDA) ^ ks0 ^ ks1
    x0 = jnp.zeros_like(x1) + ks0
    x1 = x1 + ks1

    def rounds(x0, x1, rots):
        for d in rots:
            x0 = x0 + x1
            x1 = (x1 << d) | (x1 >> (32 - d))
            x1 = x1 ^ x0
        return x0, x1

    r_a = (13, 15, 26, 6)
    r_b = (17, 29, 16, 24)
    for i, (a0, a1, rots) in enumerate(
        [(ks1, ks2, r_a), (ks2, ks0, r_b), (ks0, ks1, r_a),
         (ks1, ks2, r_b), (ks2, ks0, r_a)]
    ):
        x0, x1 = rounds(x0, x1, rots)
        x0 = x0 + a0
        x1 = x1 + a1 + jnp.uint32(i + 1)
    return x0 ^ x1


def _body(n, bs, bi_ref, bj_ref, probs_ref, out_ref, scratch_ref):
    p = pl.program_id(0)
    k = pl.program_id(1)
    bi = bi_ref[p]
    bj = bj_ref[p]

    @pl.when(k == 0)
    def _compute():
        rl = lax.broadcasted_iota(jnp.int32, (bs, bs), 0)
        cl = lax.broadcasted_iota(jnp.int32, (bs, bs), 1)
        r = rl + bi * bs
        c = cl + bj * bs
        flat = (r * n + c).astype(jnp.uint32)
        bits = _threefry_bits_u32(flat)
        fbits = (bits >> 9) | jnp.uint32(0x3F800000)
        noise = lax.bitcast_convert_type(fbits, jnp.float32) - 1.0
        bern = (noise < probs_ref[...]).astype(jnp.float32)
        bern_t = bern.T
        diag = bi == bj
        lower = rl > cl
        out_ref[...] = jnp.where(diag & lower, bern_t, bern)
        scratch_ref[...] = jnp.where(diag & jnp.logical_not(lower), bern, bern_t)

    @pl.when(k == 1)
    def _mirror():
        out_ref[...] = scratch_ref[...]


def kernel(edge_probs):
    n = edge_probs.shape[0]
    bs = _BS
    nb = n // bs
    pairs = [(i, j) for i in range(nb) for j in range(i, nb)]
    bi_arr = jnp.asarray(np.array([ij[0] for ij in pairs], dtype=np.int32))
    bj_arr = jnp.asarray(np.array([ij[1] for ij in pairs], dtype=np.int32))
    num_pairs = len(pairs)

    grid_spec = pltpu.PrefetchScalarGridSpec(
        num_scalar_prefetch=2,
        grid=(num_pairs, 2),
        in_specs=[
            pl.BlockSpec((bs, bs), lambda p, k, bi, bj: (bi[p], bj[p])),
        ],
        out_specs=pl.BlockSpec(
            (bs, bs),
            lambda p, k, bi, bj: (
                jnp.where(k == 0, bi[p], bj[p]),
                jnp.where(k == 0, bj[p], bi[p]),
            ),
        ),
        scratch_shapes=[pltpu.VMEM((bs, bs), jnp.float32)],
    )
    return pl.pallas_call(
        functools.partial(_body, n, bs),
        grid_spec=grid_spec,
        out_shape=jax.ShapeDtypeStruct((n, n), jnp.float32),
        compiler_params=pltpu.CompilerParams(
            dimension_semantics=("parallel", "arbitrary"),
        ),
    )(bi_arr, bj_arr, edge_probs)


# speedup vs baseline: 1.9498x; 1.2462x over previous
import functools

import jax
import jax.numpy as jnp
import numpy as np
from jax import lax
from jax.experimental import pallas as pl
from jax.experimental.pallas import tpu as pltpu

_BS = 1024


def _threefry_bits_u32(x1):
    ks0 = jnp.uint32(0)
    ks1 = jnp.uint32(42)
    ks2 = jnp.uint32(0x1BD11BDA) ^ ks0 ^ ks1
    x0 = jnp.zeros_like(x1) + ks0
    x1 = x1 + ks1

    def rounds(x0, x1, rots):
        for d in rots:
            x0 = x0 + x1
            x1 = (x1 << d) | (x1 >> (32 - d))
            x1 = x1 ^ x0
        return x0, x1

    r_a = (13, 15, 26, 6)
    r_b = (17, 29, 16, 24)
    for i, (a0, a1, rots) in enumerate(
        [(ks1, ks2, r_a), (ks2, ks0, r_b), (ks0, ks1, r_a),
         (ks1, ks2, r_b), (ks2, ks0, r_a)]
    ):
        x0, x1 = rounds(x0, x1, rots)
        x0 = x0 + a0
        x1 = x1 + a1 + jnp.uint32(i + 1)
    return x0 ^ x1


def _body(n, bs, num_pairs, bi_ref, bj_ref, probs_ref, out_ref,
          up_buf, tr_buf, sems):
    p = pl.program_id(0)
    slot = lax.rem(p, 2)
    bi = bi_ref[p]
    bj = bj_ref[p]

    def copy_desc(buf, row_blk, col_blk, sem_idx):
        return pltpu.make_async_copy(
            buf.at[slot],
            out_ref.at[pl.ds(row_blk * bs, bs), pl.ds(col_blk * bs, bs)],
            sems.at[slot, sem_idx],
        )

    @pl.when(p >= 2)
    def _drain_prev():
        copy_desc(up_buf, bi, bj, 0).wait()
        copy_desc(tr_buf, bj, bi, 1).wait()

    rl = lax.broadcasted_iota(jnp.int32, (bs, bs), 0)
    cl = lax.broadcasted_iota(jnp.int32, (bs, bs), 1)
    r = rl + bi * bs
    c = cl + bj * bs
    flat = (r * n + c).astype(jnp.uint32)
    bits = _threefry_bits_u32(flat)
    fbits = (bits >> 9) | jnp.uint32(0x3F800000)
    noise = lax.bitcast_convert_type(fbits, jnp.float32) - 1.0
    bern = (noise < probs_ref[...]).astype(jnp.float32)
    bern_t = bern.T
    diag = bi == bj
    lower = rl > cl
    up_buf[slot] = jnp.where(diag & lower, bern_t, bern)
    tr_buf[slot] = jnp.where(diag & jnp.logical_not(lower), bern, bern_t)

    copy_desc(up_buf, bi, bj, 0).start()
    copy_desc(tr_buf, bj, bi, 1).start()

    @pl.when(p == num_pairs - 1)
    def _drain_tail():
        copy_desc(up_buf, bi, bj, 0).wait()
        copy_desc(tr_buf, bj, bi, 1).wait()
        if num_pairs >= 2:
            other = 1 - slot
            pltpu.make_async_copy(
                up_buf.at[other],
                out_ref.at[pl.ds(0, bs), pl.ds(0, bs)],
                sems.at[other, 0],
            ).wait()
            pltpu.make_async_copy(
                tr_buf.at[other],
                out_ref.at[pl.ds(0, bs), pl.ds(0, bs)],
                sems.at[other, 1],
            ).wait()


def kernel(edge_probs):
    n = edge_probs.shape[0]
    bs = _BS
    nb = n // bs
    pairs = [(i, j) for i in range(nb) for j in range(i, nb)]
    bi_arr = jnp.asarray(np.array([ij[0] for ij in pairs], dtype=np.int32))
    bj_arr = jnp.asarray(np.array([ij[1] for ij in pairs], dtype=np.int32))
    num_pairs = len(pairs)

    grid_spec = pltpu.PrefetchScalarGridSpec(
        num_scalar_prefetch=2,
        grid=(num_pairs,),
        in_specs=[
            pl.BlockSpec((bs, bs), lambda p, bi, bj: (bi[p], bj[p])),
        ],
        out_specs=pl.BlockSpec(memory_space=pl.ANY),
        scratch_shapes=[
            pltpu.VMEM((2, bs, bs), jnp.float32),
            pltpu.VMEM((2, bs, bs), jnp.float32),
            pltpu.SemaphoreType.DMA((2, 2)),
        ],
    )
    return pl.pallas_call(
        functools.partial(_body, n, bs, num_pairs),
        grid_spec=grid_spec,
        out_shape=jax.ShapeDtypeStruct((n, n), jnp.float32),
        compiler_params=pltpu.CompilerParams(
            dimension_semantics=("arbitrary",),
        ),
    )(bi_arr, bj_arr, edge_probs)


# speedup vs baseline: 2.0286x; 1.0404x over previous
import functools

import jax
import jax.numpy as jnp
import numpy as np
from jax import lax
from jax.experimental import pallas as pl
from jax.experimental.pallas import tpu as pltpu

_BS = 1024


def _threefry_bits_u32(x1):
    ks0 = jnp.uint32(0)
    ks1 = jnp.uint32(42)
    ks2 = jnp.uint32(0x1BD11BDA) ^ ks0 ^ ks1
    x0 = jnp.zeros_like(x1) + ks0

    def rounds(x0, x1, rots):
        for d in rots:
            x0 = x0 + x1
            x1 = (x1 << d) | (x1 >> (32 - d))
            x1 = x1 ^ x0
        return x0, x1

    r_a = (13, 15, 26, 6)
    r_b = (17, 29, 16, 24)
    for i, (a0, a1, rots) in enumerate(
        [(ks1, ks2, r_a), (ks2, ks0, r_b), (ks0, ks1, r_a),
         (ks1, ks2, r_b), (ks2, ks0, r_a)]
    ):
        x0, x1 = rounds(x0, x1, rots)
        x0 = x0 + a0
        x1 = x1 + a1 + jnp.uint32(i + 1)
    return x0 ^ x1


def _body(n, bs, num_pairs, bi_ref, bj_ref, probs_ref, out_ref,
          up_buf, tr_buf, iota_buf, sems):
    p = pl.program_id(0)
    slot = lax.rem(p, 2)
    bi = bi_ref[p]
    bj = bj_ref[p]

    @pl.when(p == 0)
    def _init_iota():
        rl = lax.broadcasted_iota(jnp.uint32, (bs, bs), 0)
        cl = lax.broadcasted_iota(jnp.uint32, (bs, bs), 1)
        iota_buf[...] = rl * jnp.uint32(n) + cl

    def copy_desc(buf, row_blk, col_blk, sem_idx):
        return pltpu.make_async_copy(
            buf.at[slot],
            out_ref.at[pl.ds(row_blk * bs, bs), pl.ds(col_blk * bs, bs)],
            sems.at[slot, sem_idx],
        )

    @pl.when(p >= 2)
    def _drain_prev():
        copy_desc(up_buf, bi, bj, 0).wait()
        copy_desc(tr_buf, bj, bi, 1).wait()

    base = (bi * (bs * n) + bj * bs + 42).astype(jnp.uint32)
    bits = _threefry_bits_u32(iota_buf[...] + base)
    fbits = (bits >> 9) | jnp.uint32(0x3F800000)
    noise = lax.bitcast_convert_type(fbits, jnp.float32) - 1.0
    bern = (noise < probs_ref[...]).astype(jnp.float32)
    bern_t = bern.T
    diag = bi == bj

    @pl.when(jnp.logical_not(diag))
    def _off_diag():
        up_buf[slot] = bern
        tr_buf[slot] = bern_t
        copy_desc(tr_buf, bj, bi, 1).start()

    @pl.when(diag)
    def _diag():
        rl = lax.broadcasted_iota(jnp.int32, (bs, bs), 0)
        cl = lax.broadcasted_iota(jnp.int32, (bs, bs), 1)
        up_buf[slot] = jnp.where(rl > cl, bern_t, bern)
        copy_desc(up_buf, bj, bi, 1).start()

    copy_desc(up_buf, bi, bj, 0).start()

    @pl.when(p == num_pairs - 1)
    def _drain_tail():
        copy_desc(up_buf, bi, bj, 0).wait()
        copy_desc(tr_buf, bj, bi, 1).wait()
        if num_pairs >= 2:
            other = 1 - slot
            pltpu.make_async_copy(
                up_buf.at[other],
                out_ref.at[pl.ds(0, bs), pl.ds(0, bs)],
                sems.at[other, 0],
            ).wait()
            pltpu.make_async_copy(
                tr_buf.at[other],
                out_ref.at[pl.ds(0, bs), pl.ds(0, bs)],
                sems.at[other, 1],
            ).wait()


def kernel(edge_probs):
    n = edge_probs.shape[0]
    bs = _BS
    nb = n // bs
    pairs = [(i, j) for i in range(nb) for j in range(i, nb)]
    bi_arr = jnp.asarray(np.array([ij[0] for ij in pairs], dtype=np.int32))
    bj_arr = jnp.asarray(np.array([ij[1] for ij in pairs], dtype=np.int32))
    num_pairs = len(pairs)

    grid_spec = pltpu.PrefetchScalarGridSpec(
        num_scalar_prefetch=2,
        grid=(num_pairs,),
        in_specs=[
            pl.BlockSpec((bs, bs), lambda p, bi, bj: (bi[p], bj[p])),
        ],
        out_specs=pl.BlockSpec(memory_space=pl.ANY),
        scratch_shapes=[
            pltpu.VMEM((2, bs, bs), jnp.float32),
            pltpu.VMEM((2, bs, bs), jnp.float32),
            pltpu.VMEM((bs, bs), jnp.uint32),
            pltpu.SemaphoreType.DMA((2, 2)),
        ],
    )
    return pl.pallas_call(
        functools.partial(_body, n, bs, num_pairs),
        grid_spec=grid_spec,
        out_shape=jax.ShapeDtypeStruct((n, n), jnp.float32),
        compiler_params=pltpu.CompilerParams(
            dimension_semantics=("arbitrary",),
        ),
    )(bi_arr, bj_arr, edge_probs)
